# baseline (speedup 1.0000x reference)
import functools

import jax
import jax.numpy as jnp
from jax import lax
from jax.experimental import pallas as pl
from jax.experimental.pallas import tpu as pltpu
from jax.experimental.pallas import tpu_sc as plsc

NUM_EMB = 100000
EMBED_DIM = 64
BATCH = 16384

_info = plsc.get_sparse_core_info()
_NC, _NS = _info.num_cores, _info.num_subcores
_NW = _NC * _NS
_B_PER_W = BATCH // _NW
_T_PER_W = _B_PER_W // 8
_NCH = 4
_KPC = _B_PER_W // 16 // _NCH


def _gather_body(y_hbm, emb_hbm, out_hbm, idx_v, buf, stage, gsem, osem):
    wid = lax.axis_index("s") * _NC + lax.axis_index("c")
    base = wid * _B_PER_W
    pltpu.sync_copy(y_hbm.at[pl.ds(base, _B_PER_W)], idx_v)

    def fire(k, _):
        v = idx_v[pl.ds(k * 16, 16)]
        for l in range(16):
            idx = v[l]
            q = jax.lax.shift_right_logical(idx, 3)
            r = jax.lax.rem(idx, 8)
            pltpu.async_copy(
                emb_hbm.at[q, r], buf.at[2 * k + l // 8, l % 8], gsem
            )
        return 0

    lanes = jax.lax.iota(jnp.int32, 16)
    c8q = [(lanes + q * 16) // 8 for q in range(EMBED_DIM // 16)]
    c0q = [jax.lax.rem(lanes + q * 16, 8) for q in range(EMBED_DIM // 16)]

    lax.fori_loop(0, _KPC, fire, 0)
    for c in range(_NCH):
        if c + 1 < _NCH:
            lax.fori_loop((c + 1) * _KPC, (c + 2) * _KPC, fire, 0)
        pltpu.make_async_copy(
            emb_hbm.at[pl.ds(0, 16)], buf.at[pl.ds(c * 16, 16)], gsem
        ).wait()
        itv = jnp.full((16,), c, dtype=jnp.int32)

        def trow(j2, _):
            t = c * 16 + jax.lax.shift_right_logical(j2, 3)
            r = jnp.bitwise_and(j2, 7)
            il = jnp.full((16,), j2, dtype=jnp.int32)
            for q in range(EMBED_DIM // 16):
                v = buf[t, r, pl.ds(q * 16, 16)]
                plsc.store_scatter(stage, [c8q[q], itv, c0q[q], il], v)
            return 0

        lax.fori_loop(0, 128, trow, 0, unroll=4)
        for c8 in range(8):
            pltpu.async_copy(
                stage.at[c8, c], out_hbm.at[c8, wid * _NCH + c], osem
            )
    pltpu.make_async_copy(
        stage, out_hbm.at[:, pl.ds(wid * _NCH, _NCH)], osem
    ).wait()


@jax.jit
def kernel(y, emb):
    emb3 = emb.reshape(NUM_EMB // 8, 8, EMBED_DIM)
    mesh = plsc.VectorSubcoreMesh(core_axis_name="c", subcore_axis_name="s")
    f = functools.partial(
        pl.kernel,
        mesh=mesh,
        out_type=jax.ShapeDtypeStruct((8, BATCH // 128, 8, 128), jnp.float32),
        scratch_types=[
            pltpu.VMEM((_B_PER_W,), jnp.int32),
            pltpu.VMEM((_T_PER_W, 8, EMBED_DIM), jnp.float32),
            pltpu.VMEM((8, _NCH, 8, 128), jnp.float32),
            pltpu.SemaphoreType.DMA,
            pltpu.SemaphoreType.DMA,
        ],
        compiler_params=pltpu.CompilerParams(needs_layout_passes=False),
    )(_gather_body)
    out4 = f(y, emb3)
    return out4.transpose(1, 3, 0, 2).reshape(BATCH, EMBED_DIM)


# speedup vs baseline: 1.2213x; 1.2213x over previous
import functools

import jax
import jax.numpy as jnp
from jax import lax
from jax.experimental import pallas as pl
from jax.experimental.pallas import tpu as pltpu
from jax.experimental.pallas import tpu_sc as plsc

NUM_EMB = 100000
EMBED_DIM = 64
BATCH = 16384

_info = plsc.get_sparse_core_info()
_NC, _NS = _info.num_cores, _info.num_subcores
_NW = _NC * _NS
_B_PER_W = BATCH // _NW
_T_PER_W = _B_PER_W // 8


def _gather_body(y_hbm, emb_hbm, out_hbm, idx_v, buf, gsem):
    wid = lax.axis_index("s") * _NC + lax.axis_index("c")
    base = wid * _B_PER_W
    pltpu.sync_copy(y_hbm.at[pl.ds(base, _B_PER_W)], idx_v)

    def body(k, _):
        v = idx_v[pl.ds(k * 16, 16)]
        for l in range(16):
            idx = v[l]
            q = jax.lax.shift_right_logical(idx, 3)
            r = jax.lax.rem(idx, 8)
            pltpu.async_copy(
                emb_hbm.at[q, r], buf.at[2 * k + l // 8, l % 8], gsem
            )
        return 0

    lax.fori_loop(0, _B_PER_W // 16, body, 0)
    pltpu.make_async_copy(emb_hbm.at[pl.ds(0, _T_PER_W)], buf, gsem).wait()
    pltpu.sync_copy(buf, out_hbm.at[pl.ds(wid * _T_PER_W, _T_PER_W)])


@jax.jit
def kernel(y, emb):
    emb3 = emb.reshape(NUM_EMB // 8, 8, EMBED_DIM)
    mesh = plsc.VectorSubcoreMesh(core_axis_name="c", subcore_axis_name="s")
    f = functools.partial(
        pl.kernel,
        mesh=mesh,
        out_type=jax.ShapeDtypeStruct((BATCH // 8, 8, EMBED_DIM), jnp.float32),
        scratch_types=[
            pltpu.VMEM((_B_PER_W,), jnp.int32),
            pltpu.VMEM((_T_PER_W, 8, EMBED_DIM), jnp.float32),
            pltpu.SemaphoreType.DMA,
        ],
    )(_gather_body)
    out3 = f(y, emb3)
    return out3.reshape(BATCH, EMBED_DIM)
